# baseline (speedup 1.0000x reference)
import jax
import jax.numpy as jnp
from jax import lax
from jax.experimental import pallas as pl
from jax.experimental.pallas import tpu as pltpu
from jax.experimental.pallas import tpu_sc as plsc

N = 10000
E = 320000
D = 128
NC = 2
NS = 16
C = 128
TILES = NC * NS
CHUNKS_PER_TILE = 80
E_PAD = TILES * CHUNKS_PER_TILE * C
ROWS_A = 624
ROWS_REM = N - NS * ROWS_A
_BCAST_DNUMS = lax.GatherDimensionNumbers(
    offset_dims=(), collapsed_slice_dims=(0,), start_index_map=(0,))


def _bcast_lane(v16, l):
    idx = jnp.full((16, 1), l, dtype=jnp.int32)
    return lax.gather(v16, idx, _BCAST_DNUMS, (1,),
                      mode=lax.GatherScatterMode.PROMISE_IN_BOUNDS)


def _scale_rows(rows_v, w_c):
    def grp(g, carry):
        wf = w_c[pl.ds(g * 16, 16)]
        for l in range(16):
            wl = _bcast_lane(wf, l)
            r = g * 16 + l
            for j in range(D // 16):
                sl = pl.ds(j * 16, 16)
                rows_v[r, sl] = rows_v[r, sl] * wl
        return carry

    lax.fori_loop(0, C // 16, grp, 0)


def _spmm_body(feat, col3, w3, row3, out, row_t, col0, col1, w0, w1,
               rows0, rows1, acc, sg0, sg1, ss0, ss1):
    c = lax.axis_index("c")
    s = lax.axis_index("s")
    tile = s * NC + c

    pltpu.sync_copy(row3.at[tile], row_t)
    pltpu.sync_copy(col3.at[tile, 0], col0)
    pltpu.sync_copy(w3.at[tile, 0], w0)
    pltpu.async_copy(feat.at[col0], rows0, sg0)

    z = jnp.zeros((16,), jnp.float32)

    def zrow(r, carry):
        for j in range(D // 16):
            rows1[r, pl.ds(j * 16, 16)] = z
        return carry

    lax.fori_loop(0, C, zrow, 0)
    base_rows = s * ROWS_A
    nfull = ROWS_A // C
    for i in range(nfull):
        pltpu.sync_copy(rows1, acc.at[pl.ds(base_rows + i * C, C)])
    rem = ROWS_A - nfull * C
    pltpu.sync_copy(rows1.at[pl.ds(0, rem)],
                    acc.at[pl.ds(base_rows + nfull * C, rem)])

    @pl.when(s == NS - 1)
    def _zero_tail():
        pltpu.sync_copy(rows1.at[pl.ds(0, ROWS_REM)],
                        acc.at[pl.ds(NS * ROWS_A, ROWS_REM)])

    plsc.subcore_barrier()

    def chunk2(k, carry):
        e = 2 * k

        @pl.when(k > 0)
        def _drain_s1():
            pltpu.make_async_copy(rows1, acc.at[row_t.at[e - 1]], ss1).wait()

        pltpu.sync_copy(col3.at[tile, e + 1], col1)
        pltpu.sync_copy(w3.at[tile, e + 1], w1)
        pltpu.async_copy(feat.at[col1], rows1, sg1)

        pltpu.make_async_copy(feat.at[col0], rows0, sg0).wait()
        _scale_rows(rows0, w0)
        pltpu.async_copy(rows0, acc.at[row_t.at[e]], ss0, add=True)

        pltpu.make_async_copy(feat.at[col1], rows1, sg1).wait()
        _scale_rows(rows1, w1)

        pltpu.make_async_copy(rows0, acc.at[row_t.at[e]], ss0).wait()

        @pl.when(k < CHUNKS_PER_TILE // 2 - 1)
        def _prefetch0():
            pltpu.sync_copy(col3.at[tile, e + 2], col0)
            pltpu.sync_copy(w3.at[tile, e + 2], w0)
            pltpu.async_copy(feat.at[col0], rows0, sg0)

        pltpu.async_copy(rows1, acc.at[row_t.at[e + 1]], ss1, add=True)
        return carry

    lax.fori_loop(0, CHUNKS_PER_TILE // 2, chunk2, 0)
    pltpu.make_async_copy(rows1, acc.at[row_t.at[CHUNKS_PER_TILE - 1]],
                          ss1).wait()
    plsc.subcore_barrier()

    pltpu.sync_copy(acc.at[pl.ds(base_rows, ROWS_A)],
                    out.at[c, pl.ds(base_rows, ROWS_A)])

    @pl.when(s == NS - 1)
    def _write_tail():
        pltpu.sync_copy(acc.at[pl.ds(NS * ROWS_A, ROWS_REM)],
                        out.at[c, pl.ds(NS * ROWS_A, ROWS_REM)])


def _make_spmm():
    mesh = plsc.VectorSubcoreMesh(core_axis_name="c", subcore_axis_name="s",
                                  num_cores=NC, num_subcores=NS)
    return pl.kernel(
        _spmm_body,
        out_type=jax.ShapeDtypeStruct((NC, N, D), jnp.float32),
        mesh=mesh,
        scratch_types=[
            pltpu.VMEM((CHUNKS_PER_TILE, C), jnp.int32),
            pltpu.VMEM((C,), jnp.int32),
            pltpu.VMEM((C,), jnp.int32),
            pltpu.VMEM((C,), jnp.float32),
            pltpu.VMEM((C,), jnp.float32),
            pltpu.VMEM((C, D), jnp.float32),
            pltpu.VMEM((C, D), jnp.float32),
            pltpu.VMEM_SHARED((N, D), jnp.float32),
            pltpu.SemaphoreType.DMA,
            pltpu.SemaphoreType.DMA,
            pltpu.SemaphoreType.DMA,
            pltpu.SemaphoreType.DMA,
        ],
    )


_ROWS_BLK = 1000


def _add_body(p_ref, o_ref):
    o_ref[...] = p_ref[0] + p_ref[1]


def _h1_add(p):
    return pl.pallas_call(
        _add_body,
        out_shape=jax.ShapeDtypeStruct((N, D), jnp.float32),
        grid=(N // _ROWS_BLK,),
        in_specs=[pl.BlockSpec((NC, _ROWS_BLK, D), lambda i: (0, i, 0))],
        out_specs=pl.BlockSpec((_ROWS_BLK, D), lambda i: (i, 0)),
    )(p)


def _final_body(x_ref, h1_ref, p2_ref, wt_ref, b_ref, o_ref):
    h2 = p2_ref[0] + p2_ref[1]
    acc = jnp.dot(x_ref[...], wt_ref[0:D], preferred_element_type=jnp.float32)
    acc = acc + jnp.dot(h1_ref[...], wt_ref[D:2 * D],
                        preferred_element_type=jnp.float32)
    acc = acc + jnp.dot(h2, wt_ref[2 * D:3 * D],
                        preferred_element_type=jnp.float32)
    o_ref[...] = acc + b_ref[...]


def _final(x, h1, p2, Wt, b2):
    return pl.pallas_call(
        _final_body,
        out_shape=jax.ShapeDtypeStruct((N, D), jnp.float32),
        grid=(N // _ROWS_BLK,),
        in_specs=[
            pl.BlockSpec((_ROWS_BLK, D), lambda i: (i, 0)),
            pl.BlockSpec((_ROWS_BLK, D), lambda i: (i, 0)),
            pl.BlockSpec((NC, _ROWS_BLK, D), lambda i: (0, i, 0)),
            pl.BlockSpec((3 * D, D), lambda i: (0, 0)),
            pl.BlockSpec((1, D), lambda i: (0, 0)),
        ],
        out_specs=pl.BlockSpec((_ROWS_BLK, D), lambda i: (i, 0)),
    )(x, h1, p2, Wt, b2)


def kernel(x, edge_index, edge_weight, W, b):
    pad = E_PAD - E
    shape3 = (TILES, CHUNKS_PER_TILE, C)
    row = jnp.concatenate(
        [edge_index[0], jnp.zeros((pad,), jnp.int32)]).reshape(shape3)
    col = jnp.concatenate(
        [edge_index[1], jnp.zeros((pad,), jnp.int32)]).reshape(shape3)
    w2 = jnp.concatenate(
        [edge_weight, jnp.zeros((pad,), jnp.float32)]).reshape(shape3)
    spmm = _make_spmm()
    p1 = spmm(x, col, w2, row)
    h1 = _h1_add(p1)
    p2 = spmm(h1, col, w2, row)
    return _final(x, h1, p2, W.T, b.reshape(1, D))


# speedup vs baseline: 1.2587x; 1.2587x over previous
import jax
import jax.numpy as jnp
from jax import lax
from jax.experimental import pallas as pl
from jax.experimental.pallas import tpu as pltpu
from jax.experimental.pallas import tpu_sc as plsc

N = 10000
E = 320000
D = 128
NC = 2
NS = 16
C = 128
TILES = NC * NS
CHUNKS_PER_TILE = 80
E_PAD = TILES * CHUNKS_PER_TILE * C
ROWS_A = 624
ROWS_REM = N - NS * ROWS_A
_BCAST_DNUMS = lax.GatherDimensionNumbers(
    offset_dims=(), collapsed_slice_dims=(0,), start_index_map=(0,))


def _bcast_lane(v16, l):
    idx = jnp.full((16, 1), l, dtype=jnp.int32)
    return lax.gather(v16, idx, _BCAST_DNUMS, (1,),
                      mode=lax.GatherScatterMode.PROMISE_IN_BOUNDS)


def _scale_rows(rows_v, w_t, f):
    def grp(g, carry):
        wf = w_t[f, pl.ds(g * 16, 16)]
        for l in range(16):
            wl = _bcast_lane(wf, l)
            r = g * 16 + l
            for j in range(D // 16):
                sl = pl.ds(j * 16, 16)
                rows_v[r, sl] = rows_v[r, sl] * wl
        return carry

    lax.fori_loop(0, C // 16, grp, 0)


def _spmm_body(feat, idx3, w3, out, w_t, i0, i1, i2, i3,
               rows0, rows1, acc, sg0, sg1, ss0, ss1, si0, si1, si2, si3):
    c = lax.axis_index("c")
    s = lax.axis_index("s")
    tile = s * NC + c
    rows = (rows0, rows1)
    idx = (i0, i1, i2, i3)
    sg = (sg0, sg1)
    ss = (ss0, ss1)
    si = (si0, si1, si2, si3)

    pltpu.sync_copy(w3.at[tile], w_t)
    pltpu.sync_copy(idx3.at[tile, 0], i0)
    pltpu.sync_copy(idx3.at[tile, 1], i1)
    pltpu.async_copy(feat.at[i0.at[0]], rows0, sg0)

    z = jnp.zeros((16,), jnp.float32)

    def zrow(r, carry):
        for j in range(D // 16):
            rows1[r, pl.ds(j * 16, 16)] = z
        return carry

    lax.fori_loop(0, C, zrow, 0)
    base_rows = s * ROWS_A
    nfull = ROWS_A // C
    for i in range(nfull):
        pltpu.sync_copy(rows1, acc.at[pl.ds(base_rows + i * C, C)])
    rem = ROWS_A - nfull * C
    pltpu.sync_copy(rows1.at[pl.ds(0, rem)],
                    acc.at[pl.ds(base_rows + nfull * C, rem)])

    @pl.when(s == NS - 1)
    def _zero_tail():
        pltpu.sync_copy(rows1.at[pl.ds(0, ROWS_REM)],
                        acc.at[pl.ds(NS * ROWS_A, ROWS_REM)])

    plsc.subcore_barrier()

    def quad(K, carry):
        f0 = 4 * K
        for t in range(4):
            f = f0 + t
            p = t % 2
            q = (t + 1) % 2
            jn = (t + 1) % 4
            jp = (t + 2) % 4

            def _launch_next():
                pltpu.make_async_copy(rows[q], acc.at[idx[jn].at[1]],
                                      ss[q]).wait()
                pltpu.make_async_copy(idx3.at[tile, f + 1], idx[jn],
                                      si[jn]).wait()
                pltpu.async_copy(feat.at[idx[jn].at[0]], rows[q], sg[q])

            def _launch_next_nodrain():
                pltpu.async_copy(feat.at[idx[jn].at[0]], rows[q], sg[q])

            if t == 0:
                @pl.when(K > 0)
                def _ln():
                    _launch_next()

                @pl.when(K == 0)
                def _ln0():
                    _launch_next_nodrain()
            elif t == 3:
                @pl.when(K < CHUNKS_PER_TILE // 4 - 1)
                def _ln3():
                    _launch_next()
            else:
                if t == 1:
                    pltpu.make_async_copy(rows[q], acc.at[idx[jn].at[1]],
                                          ss[q]).wait()
                    pltpu.make_async_copy(idx3.at[tile, f + 1], idx[jn],
                                          si[jn]).wait()
                    pltpu.async_copy(feat.at[idx[jn].at[0]], rows[q], sg[q])
                else:
                    _launch_next()

            pltpu.make_async_copy(feat.at[idx[t].at[0]], rows[p], sg[p]).wait()
            _scale_rows(rows[p], w_t, f)
            pltpu.async_copy(rows[p], acc.at[idx[t].at[1]], ss[p], add=True)

            @pl.when(f + 2 < CHUNKS_PER_TILE)
            def _pf():
                pltpu.async_copy(idx3.at[tile, f + 2], idx[jp], si[jp])

        return carry

    lax.fori_loop(0, CHUNKS_PER_TILE // 4, quad, 0)
    pltpu.make_async_copy(rows1, acc.at[i3.at[1]], ss1).wait()
    plsc.subcore_barrier()

    pltpu.sync_copy(acc.at[pl.ds(base_rows, ROWS_A)],
                    out.at[c, pl.ds(base_rows, ROWS_A)])

    @pl.when(s == NS - 1)
    def _write_tail():
        pltpu.sync_copy(acc.at[pl.ds(NS * ROWS_A, ROWS_REM)],
                        out.at[c, pl.ds(NS * ROWS_A, ROWS_REM)])


def _make_spmm():
    mesh = plsc.VectorSubcoreMesh(core_axis_name="c", subcore_axis_name="s",
                                  num_cores=NC, num_subcores=NS)
    return pl.kernel(
        _spmm_body,
        out_type=jax.ShapeDtypeStruct((NC, N, D), jnp.float32),
        mesh=mesh,
        scratch_types=[
            pltpu.VMEM((CHUNKS_PER_TILE, C), jnp.float32),
            pltpu.VMEM((2, C), jnp.int32),
            pltpu.VMEM((2, C), jnp.int32),
            pltpu.VMEM((2, C), jnp.int32),
            pltpu.VMEM((2, C), jnp.int32),
            pltpu.VMEM((C, D), jnp.float32),
            pltpu.VMEM((C, D), jnp.float32),
            pltpu.VMEM_SHARED((N, D), jnp.float32),
        ] + [pltpu.SemaphoreType.DMA] * 8,
    )


_ROWS_BLK = 1000


def _add_body(p_ref, o_ref):
    o_ref[...] = p_ref[0] + p_ref[1]


def _h1_add(p):
    return pl.pallas_call(
        _add_body,
        out_shape=jax.ShapeDtypeStruct((N, D), jnp.float32),
        grid=(N // _ROWS_BLK,),
        in_specs=[pl.BlockSpec((NC, _ROWS_BLK, D), lambda i: (0, i, 0))],
        out_specs=pl.BlockSpec((_ROWS_BLK, D), lambda i: (i, 0)),
    )(p)


def _final_body(x_ref, h1_ref, p2_ref, wt_ref, b_ref, o_ref):
    h2 = p2_ref[0] + p2_ref[1]
    acc = jnp.dot(x_ref[...], wt_ref[0:D], preferred_element_type=jnp.float32)
    acc = acc + jnp.dot(h1_ref[...], wt_ref[D:2 * D],
                        preferred_element_type=jnp.float32)
    acc = acc + jnp.dot(h2, wt_ref[2 * D:3 * D],
                        preferred_element_type=jnp.float32)
    o_ref[...] = acc + b_ref[...]


def _final(x, h1, p2, Wt, b2):
    return pl.pallas_call(
        _final_body,
        out_shape=jax.ShapeDtypeStruct((N, D), jnp.float32),
        grid=(N // _ROWS_BLK,),
        in_specs=[
            pl.BlockSpec((_ROWS_BLK, D), lambda i: (i, 0)),
            pl.BlockSpec((_ROWS_BLK, D), lambda i: (i, 0)),
            pl.BlockSpec((NC, _ROWS_BLK, D), lambda i: (0, i, 0)),
            pl.BlockSpec((3 * D, D), lambda i: (0, 0)),
            pl.BlockSpec((1, D), lambda i: (0, 0)),
        ],
        out_specs=pl.BlockSpec((_ROWS_BLK, D), lambda i: (i, 0)),
    )(x, h1, p2, Wt, b2)


def kernel(x, edge_index, edge_weight, W, b):
    pad = E_PAD - E
    shape3 = (TILES, CHUNKS_PER_TILE, C)
    row = jnp.concatenate(
        [edge_index[0], jnp.zeros((pad,), jnp.int32)]).reshape(shape3)
    col = jnp.concatenate(
        [edge_index[1], jnp.zeros((pad,), jnp.int32)]).reshape(shape3)
    w2 = jnp.concatenate(
        [edge_weight, jnp.zeros((pad,), jnp.float32)]).reshape(shape3)
    idx3 = jnp.stack([col, row], axis=2)
    spmm = _make_spmm()
    p1 = spmm(x, idx3, w2)
    h1 = _h1_add(p1)
    p2 = spmm(h1, idx3, w2)
    return _final(x, h1, p2, W.T, b.reshape(1, D))
